# baseline (speedup 1.0000x reference)
import functools

import jax
import jax.numpy as jnp
import numpy as np
from jax import lax
from jax.experimental import pallas as pl
from jax.experimental.pallas import tpu as pltpu
from jax.experimental.pallas import tpu_sc as plsc

N = 10000
E = 320000
D = 128
DE = 16
H = 128
G = 64

NC = 2
NS = 16
L = 16
NW = NC * NS
EPW = E // NW
CHUNK = 40
NCH = EPW // CHUNK
ROWS_PT = 632
NP = ROWS_PT * NS
VPR = H // L
NBUF = 3
NIB = 6
GRP = 6
NMAIN = (NCH // GRP) * GRP


def _mp_body(node_hbm, eemb_hbm, ei_hbm, out_hbm,
             srcr, dstr, rows2, ee2, msg2, agg_sh,
             sem_i, sem_g, sem_e, sem_s):
    c = lax.axis_index("c")
    s = lax.axis_index("s")
    wid = s * NC + c
    ebase = wid * EPW

    def start_idx(k, si):
        pltpu.async_copy(ei_hbm.at[0, wid, k], srcr[si], sem_i[si])
        pltpu.async_copy(ei_hbm.at[1, wid, k], dstr[si], sem_i[si])

    def wait_idx(k, si):
        pltpu.make_async_copy(ei_hbm.at[0, wid, k], srcr[si], sem_i[si]).wait()
        pltpu.make_async_copy(ei_hbm.at[1, wid, k], dstr[si], sem_i[si]).wait()

    def start_loads(k, b, si):
        pltpu.async_copy(node_hbm.at[srcr[si]], rows2.at[b],
                         sem_g[b])
        pltpu.async_copy(eemb_hbm.at[pl.ds(ebase + k * CHUNK, CHUNK)],
                         ee2.at[b], sem_e[b])

    def wait_ge(k, b, si):
        pltpu.make_async_copy(node_hbm.at[srcr[si]], rows2.at[b],
                              sem_g[b]).wait()
        pltpu.make_async_copy(eemb_hbm.at[pl.ds(ebase + k * CHUNK, CHUNK)],
                              ee2.at[b], sem_e[b]).wait()

    def multiply(b):
        @plsc.parallel_loop(0, CHUNK, unroll=8)
        def _mul(i):
            for j in range(VPR):
                sl = pl.ds(j * L, L)
                msg2[b, i, sl] = rows2[b, i, sl] * ee2[b, i, sl]

    def start_scatter(b, si):
        pltpu.async_copy(msg2.at[b], agg_sh.at[dstr[si]],
                         sem_s[b], add=True)

    def wait_s(b, si):
        pltpu.make_async_copy(msg2.at[b], agg_sh.at[dstr[si]],
                              sem_s[b]).wait()

    for k in (0, 1, 2):
        start_idx(k, k)

    @pl.loop(0, CHUNK)
    def _zero(i):
        for j in range(VPR):
            rows2[0, i, pl.ds(j * L, L)] = jnp.zeros((L,), jnp.float32)

    row0 = s * ROWS_PT
    off = 0
    for sz in [CHUNK] * (ROWS_PT // CHUNK) + [ROWS_PT % CHUNK]:
        pltpu.sync_copy(rows2.at[0, pl.ds(0, sz)], agg_sh.at[pl.ds(row0 + off, sz)])
        off += sz
    plsc.subcore_barrier()

    wait_idx(0, 0)
    start_loads(0, 0, 0)
    wait_idx(1, 1)
    start_loads(1, 1, 1)

    @pl.loop(0, NMAIN // GRP)
    def _group(g):
        k0 = g * GRP
        for j in range(GRP):
            k = k0 + j
            b = j % NBUF
            start_idx(k + 3, (j + 3) % NIB)
            wait_ge(k, b, j)
            multiply(b)
            start_scatter(b, j)
            if j == 0:
                @pl.when(k >= 1)
                def _():
                    wait_s((j - 1) % NBUF, (j - 1) % NIB)
            else:
                wait_s((j - 1) % NBUF, (j - 1) % NIB)
            wait_idx(k + 2, (j + 2) % NIB)
            start_loads(k + 2, (j + 2) % NBUF, (j + 2) % NIB)

    for k in range(NMAIN, NCH):
        b = k % NBUF
        if k + 3 < NCH:
            start_idx(k + 3, (k + 3) % NIB)
        wait_ge(k, b, k % NIB)
        multiply(b)
        start_scatter(b, k % NIB)
        wait_s((k - 1) % NBUF, (k - 1) % NIB)
        if k + 2 < NCH:
            wait_idx(k + 2, (k + 2) % NIB)
            start_loads(k + 2, (k + 2) % NBUF, (k + 2) % NIB)
    wait_s((NCH - 1) % NBUF, (NCH - 1) % NIB)

    plsc.subcore_barrier()
    off = 0
    for sz in (160, 160, 160, 152):
        pltpu.sync_copy(agg_sh.at[pl.ds(row0 + off, sz)],
                        out_hbm.at[c, pl.ds(row0 + off, sz)])
        off += sz


_mp_sc = functools.partial(
    pl.kernel,
    mesh=plsc.VectorSubcoreMesh(
        core_axis_name="c", subcore_axis_name="s", num_cores=NC, num_subcores=NS),
    out_type=jax.ShapeDtypeStruct((NC, NP, H), jnp.float32),
    scratch_types=[
        [pltpu.VMEM((CHUNK,), jnp.int32)] * NIB,
        [pltpu.VMEM((CHUNK,), jnp.int32)] * NIB,
        pltpu.VMEM((NBUF, CHUNK, H), jnp.float32),
        pltpu.VMEM((NBUF, CHUNK, H), jnp.float32),
        pltpu.VMEM((NBUF, CHUNK, H), jnp.float32),
        pltpu.VMEM_SHARED((NP, H), jnp.float32),
        [pltpu.SemaphoreType.DMA] * NIB,
        [pltpu.SemaphoreType.DMA] * NBUF,
        [pltpu.SemaphoreType.DMA] * NBUF,
        [pltpu.SemaphoreType.DMA] * NBUF,
    ],
)(_mp_body)


def _embed_nodes_body(x_ref, w_ref, b_ref, o_ref):
    o_ref[...] = jnp.dot(x_ref[...], w_ref[...],
                         preferred_element_type=jnp.float32) + b_ref[...]


def _embed_edges_body(a_ref, w_ref, b_ref, o_ref):
    o_ref[...] = jnp.dot(a_ref[...], w_ref[...],
                         preferred_element_type=jnp.float32) + b_ref[...]


def _combine_body(n_ref, p0_ref, p1_ref, o_ref):
    o_ref[...] = n_ref[...] + p0_ref[0] + p1_ref[0]


def _final_body(n_ref, p0_ref, p1_ref, batch_ref,
                w1_ref, b1_ref, w2_ref, b2_ref, w3_ref, o_ref):
    i = pl.program_id(0)
    h = jax.nn.relu(n_ref[...] + p0_ref[0] + p1_ref[0])
    h = jax.nn.relu(jnp.dot(h, w1_ref[...],
                            preferred_element_type=jnp.float32) + b1_ref[...])
    h = jax.nn.relu(jnp.dot(h, w2_ref[...],
                            preferred_element_type=jnp.float32) + b2_ref[...])
    e = jnp.dot(h, w3_ref[...], preferred_element_type=jnp.float32)
    gids = lax.broadcasted_iota(jnp.int32, (1, G), 1)
    onehot = (batch_ref[...] == gids).astype(jnp.float32)
    contrib = jnp.sum(onehot * e, axis=0)

    @pl.when(i == 0)
    def _():
        o_ref[...] = jnp.zeros_like(o_ref)

    o_ref[...] += contrib[None, :]


BN = 1000
BE = 4000


def kernel(x, edge_index, edge_attr, batch, W_atom, b_atom, W_bond, b_bond,
           W1, b1, W2, b2, W3):
    ei = edge_index.astype(jnp.int32).reshape(2, NW, NCH, CHUNK)
    batch_i = batch.astype(jnp.int32).reshape(N, 1)

    full = lambda *_: (0, 0)
    node_spec = pl.BlockSpec((BN, H), lambda i: (i, 0))
    p0_spec = pl.BlockSpec((1, BN, H), lambda i: (0, i, 0))
    p1_spec = pl.BlockSpec((1, BN, H), lambda i: (1, i, 0))

    node_emb = pl.pallas_call(
        _embed_nodes_body,
        grid=(N // BN,),
        in_specs=[pl.BlockSpec((BN, D), lambda i: (i, 0)),
                  pl.BlockSpec((D, H), full),
                  pl.BlockSpec((1, H), full)],
        out_specs=node_spec,
        out_shape=jax.ShapeDtypeStruct((N, H), jnp.float32),
    )(x, W_atom.T, b_atom.reshape(1, H))

    edge_emb = pl.pallas_call(
        _embed_edges_body,
        grid=(E // BE,),
        in_specs=[pl.BlockSpec((BE, DE), lambda i: (i, 0)),
                  pl.BlockSpec((DE, H), full),
                  pl.BlockSpec((1, H), full)],
        out_specs=pl.BlockSpec((BE, H), lambda i: (i, 0)),
        out_shape=jax.ShapeDtypeStruct((E, H), jnp.float32),
    )(edge_attr, W_bond.T, b_bond.reshape(1, H))

    p = _mp_sc(node_emb, edge_emb, ei)
    node_emb1 = pl.pallas_call(
        _combine_body,
        grid=(N // BN,),
        in_specs=[node_spec, p0_spec, p1_spec],
        out_specs=node_spec,
        out_shape=jax.ShapeDtypeStruct((N, H), jnp.float32),
    )(node_emb, p, p)

    p2 = _mp_sc(node_emb1, edge_emb, ei)

    dg = pl.pallas_call(
        _final_body,
        grid=(N // BN,),
        in_specs=[node_spec, p0_spec, p1_spec,
                  pl.BlockSpec((BN, 1), lambda i: (i, 0)),
                  pl.BlockSpec((H, H), full),
                  pl.BlockSpec((1, H), full),
                  pl.BlockSpec((H, H // 2), full),
                  pl.BlockSpec((1, H // 2), full),
                  pl.BlockSpec((H // 2, 1), full)],
        out_specs=pl.BlockSpec((1, G), full),
        out_shape=jax.ShapeDtypeStruct((1, G), jnp.float32),
    )(node_emb1, p2, p2, batch_i,
      W1.T, b1.reshape(1, H), W2.T, b2.reshape(1, H // 2), W3.T)

    return dg.reshape(G, 1)


# speedup vs baseline: 1.0597x; 1.0597x over previous
import functools

import jax
import jax.numpy as jnp
import numpy as np
from jax import lax
from jax.experimental import pallas as pl
from jax.experimental.pallas import tpu as pltpu
from jax.experimental.pallas import tpu_sc as plsc

N = 10000
E = 320000
D = 128
DE = 16
H = 128
G = 64

NC = 2
NS = 16
L = 16
NW = NC * NS
EPW = E // NW
CHUNK = 40
NCH = EPW // CHUNK
ROWS_PT = 632
NP = ROWS_PT * NS
VPR = H // L
NBUF = 3
NIB = 6
GRP = 6
NMAIN = (NCH // GRP) * GRP


def _mp_body(node_hbm, eemb_hbm, ei_hbm, out_hbm,
             srcr, dstr, rows2, ee2, agg_sh,
             sem_i, sem_g, sem_e, sem_s):
    c = lax.axis_index("c")
    s = lax.axis_index("s")
    wid = s * NC + c
    ebase = wid * EPW

    def start_idx(k, si):
        pltpu.async_copy(ei_hbm.at[0, wid, k], srcr[si], sem_i[si])
        pltpu.async_copy(ei_hbm.at[1, wid, k], dstr[si], sem_i[si])

    def wait_idx(k, si):
        pltpu.make_async_copy(ei_hbm.at[0, wid, k], srcr[si], sem_i[si]).wait()
        pltpu.make_async_copy(ei_hbm.at[1, wid, k], dstr[si], sem_i[si]).wait()

    def start_loads(k, b, si):
        pltpu.async_copy(node_hbm.at[srcr[si]], rows2.at[b],
                         sem_g[b])
        pltpu.async_copy(eemb_hbm.at[pl.ds(ebase + k * CHUNK, CHUNK)],
                         ee2.at[b], sem_e[b])

    def wait_ge(k, b, si):
        pltpu.make_async_copy(node_hbm.at[srcr[si]], rows2.at[b],
                              sem_g[b]).wait()
        pltpu.make_async_copy(eemb_hbm.at[pl.ds(ebase + k * CHUNK, CHUNK)],
                              ee2.at[b], sem_e[b]).wait()

    def multiply(b):
        @plsc.parallel_loop(0, CHUNK, unroll=4)
        def _mul(i):
            for j in range(VPR):
                sl = pl.ds(j * L, L)
                rows2[b, i, sl] = rows2[b, i, sl] * ee2[b, i, sl]

    def start_scatter(b, si):
        pltpu.async_copy(rows2.at[b], agg_sh.at[dstr[si]],
                         sem_s[b], add=True)

    def wait_s(b, si):
        pltpu.make_async_copy(rows2.at[b], agg_sh.at[dstr[si]],
                              sem_s[b]).wait()

    for k in (0, 1, 2):
        start_idx(k, k)

    @pl.loop(0, CHUNK)
    def _zero(i):
        for j in range(VPR):
            rows2[0, i, pl.ds(j * L, L)] = jnp.zeros((L,), jnp.float32)

    row0 = s * ROWS_PT
    off = 0
    for sz in [CHUNK] * (ROWS_PT // CHUNK) + [ROWS_PT % CHUNK]:
        pltpu.sync_copy(rows2.at[0, pl.ds(0, sz)], agg_sh.at[pl.ds(row0 + off, sz)])
        off += sz
    plsc.subcore_barrier()

    wait_idx(0, 0)
    start_loads(0, 0, 0)
    wait_idx(1, 1)
    start_loads(1, 1, 1)

    @pl.loop(0, NMAIN // GRP)
    def _group(g):
        k0 = g * GRP
        for j in range(GRP):
            k = k0 + j
            b = j % NBUF
            start_idx(k + 3, (j + 3) % NIB)
            wait_ge(k, b, j)
            multiply(b)
            start_scatter(b, j)
            if j == 0:
                @pl.when(k >= 1)
                def _():
                    wait_s((j - 1) % NBUF, (j - 1) % NIB)
            else:
                wait_s((j - 1) % NBUF, (j - 1) % NIB)
            wait_idx(k + 2, (j + 2) % NIB)
            start_loads(k + 2, (j + 2) % NBUF, (j + 2) % NIB)

    for k in range(NMAIN, NCH):
        b = k % NBUF
        if k + 3 < NCH:
            start_idx(k + 3, (k + 3) % NIB)
        wait_ge(k, b, k % NIB)
        multiply(b)
        start_scatter(b, k % NIB)
        wait_s((k - 1) % NBUF, (k - 1) % NIB)
        if k + 2 < NCH:
            wait_idx(k + 2, (k + 2) % NIB)
            start_loads(k + 2, (k + 2) % NBUF, (k + 2) % NIB)
    wait_s((NCH - 1) % NBUF, (NCH - 1) % NIB)

    plsc.subcore_barrier()
    off = 0
    for sz in (160, 160, 160, 152):
        pltpu.sync_copy(agg_sh.at[pl.ds(row0 + off, sz)],
                        out_hbm.at[c, pl.ds(row0 + off, sz)])
        off += sz


_mp_sc = functools.partial(
    pl.kernel,
    mesh=plsc.VectorSubcoreMesh(
        core_axis_name="c", subcore_axis_name="s", num_cores=NC, num_subcores=NS),
    out_type=jax.ShapeDtypeStruct((NC, NP, H), jnp.float32),
    scratch_types=[
        [pltpu.VMEM((CHUNK,), jnp.int32)] * NIB,
        [pltpu.VMEM((CHUNK,), jnp.int32)] * NIB,
        pltpu.VMEM((NBUF, CHUNK, H), jnp.float32),
        pltpu.VMEM((NBUF, CHUNK, H), jnp.float32),
        pltpu.VMEM_SHARED((NP, H), jnp.float32),
        [pltpu.SemaphoreType.DMA] * NIB,
        [pltpu.SemaphoreType.DMA] * NBUF,
        [pltpu.SemaphoreType.DMA] * NBUF,
        [pltpu.SemaphoreType.DMA] * NBUF,
    ],
)(_mp_body)


def _embed_nodes_body(x_ref, w_ref, b_ref, o_ref):
    o_ref[...] = jnp.dot(x_ref[...], w_ref[...],
                         preferred_element_type=jnp.float32) + b_ref[...]


def _embed_edges_body(a_ref, w_ref, b_ref, o_ref):
    o_ref[...] = jnp.dot(a_ref[...], w_ref[...],
                         preferred_element_type=jnp.float32) + b_ref[...]


def _combine_body(n_ref, p0_ref, p1_ref, o_ref):
    o_ref[...] = n_ref[...] + p0_ref[0] + p1_ref[0]


def _final_body(n_ref, p0_ref, p1_ref, batch_ref,
                w1_ref, b1_ref, w2_ref, b2_ref, w3_ref, o_ref):
    i = pl.program_id(0)
    h = jax.nn.relu(n_ref[...] + p0_ref[0] + p1_ref[0])
    h = jax.nn.relu(jnp.dot(h, w1_ref[...],
                            preferred_element_type=jnp.float32) + b1_ref[...])
    h = jax.nn.relu(jnp.dot(h, w2_ref[...],
                            preferred_element_type=jnp.float32) + b2_ref[...])
    e = jnp.dot(h, w3_ref[...], preferred_element_type=jnp.float32)
    gids = lax.broadcasted_iota(jnp.int32, (1, G), 1)
    onehot = (batch_ref[...] == gids).astype(jnp.float32)
    contrib = jnp.sum(onehot * e, axis=0)

    @pl.when(i == 0)
    def _():
        o_ref[...] = jnp.zeros_like(o_ref)

    o_ref[...] += contrib[None, :]


BN = 1000
BE = 4000


def kernel(x, edge_index, edge_attr, batch, W_atom, b_atom, W_bond, b_bond,
           W1, b1, W2, b2, W3):
    ei = edge_index.astype(jnp.int32).reshape(2, NW, NCH, CHUNK)
    batch_i = batch.astype(jnp.int32).reshape(N, 1)

    full = lambda *_: (0, 0)
    node_spec = pl.BlockSpec((BN, H), lambda i: (i, 0))
    p0_spec = pl.BlockSpec((1, BN, H), lambda i: (0, i, 0))
    p1_spec = pl.BlockSpec((1, BN, H), lambda i: (1, i, 0))

    node_emb = pl.pallas_call(
        _embed_nodes_body,
        grid=(N // BN,),
        in_specs=[pl.BlockSpec((BN, D), lambda i: (i, 0)),
                  pl.BlockSpec((D, H), full),
                  pl.BlockSpec((1, H), full)],
        out_specs=node_spec,
        out_shape=jax.ShapeDtypeStruct((N, H), jnp.float32),
    )(x, W_atom.T, b_atom.reshape(1, H))

    edge_emb = pl.pallas_call(
        _embed_edges_body,
        grid=(E // BE,),
        in_specs=[pl.BlockSpec((BE, DE), lambda i: (i, 0)),
                  pl.BlockSpec((DE, H), full),
                  pl.BlockSpec((1, H), full)],
        out_specs=pl.BlockSpec((BE, H), lambda i: (i, 0)),
        out_shape=jax.ShapeDtypeStruct((E, H), jnp.float32),
    )(edge_attr, W_bond.T, b_bond.reshape(1, H))

    p = _mp_sc(node_emb, edge_emb, ei)
    node_emb1 = pl.pallas_call(
        _combine_body,
        grid=(N // BN,),
        in_specs=[node_spec, p0_spec, p1_spec],
        out_specs=node_spec,
        out_shape=jax.ShapeDtypeStruct((N, H), jnp.float32),
    )(node_emb, p, p)

    p2 = _mp_sc(node_emb1, edge_emb, ei)

    dg = pl.pallas_call(
        _final_body,
        grid=(N // BN,),
        in_specs=[node_spec, p0_spec, p1_spec,
                  pl.BlockSpec((BN, 1), lambda i: (i, 0)),
                  pl.BlockSpec((H, H), full),
                  pl.BlockSpec((1, H), full),
                  pl.BlockSpec((H, H // 2), full),
                  pl.BlockSpec((1, H // 2), full),
                  pl.BlockSpec((H // 2, 1), full)],
        out_specs=pl.BlockSpec((1, G), full),
        out_shape=jax.ShapeDtypeStruct((1, G), jnp.float32),
    )(node_emb1, p2, p2, batch_i,
      W1.T, b1.reshape(1, H), W2.T, b2.reshape(1, H // 2), W3.T)

    return dg.reshape(G, 1)
